# baseline (speedup 1.0000x reference)
import jax
import jax.numpy as jnp
from jax import lax
from jax.experimental import pallas as pl
from jax.experimental.pallas import tpu as pltpu
from jax.experimental.pallas import tpu_sc as plsc

NC = 2
NS = 16
LANES = 16
NW = NC * NS

CHUNK = 128


def _cf_body(uid_hbm, iid_hbm, utab_hbm, itab_hbm, out_hbm,
             uidx_v, iidx_v, ubuf, ibuf, out_v, sem_u, sem_i):
    d = utab_hbm.shape[1]
    n_chunks = uidx_v.shape[0]
    bpw = n_chunks * CHUNK
    wid = lax.axis_index("s") * NC + lax.axis_index("c")

    pltpu.sync_copy(uid_hbm.at[wid], uidx_v)
    pltpu.sync_copy(iid_hbm.at[wid], iidx_v)

    rows0 = lax.iota(jnp.int32, LANES)

    for j in range(n_chunks):
        cu = pltpu.async_copy(utab_hbm.at[uidx_v.at[j]], ubuf, sem_u)
        ci = pltpu.async_copy(itab_hbm.at[iidx_v.at[j]], ibuf, sem_i)
        cu.wait()
        ci.wait()
        for g in range(CHUNK // LANES):
            rows = rows0 + (g * LANES)

            def body(k, acc, rows=rows):
                kv = jnp.full((LANES,), 0, jnp.int32) + k
                u = plsc.load_gather(ubuf, [rows, kv])
                v = plsc.load_gather(ibuf, [rows, kv])
                return acc + u * v

            acc = lax.fori_loop(0, d, body, jnp.zeros((LANES,), jnp.float32))
            out_v[pl.ds(j * CHUNK + g * LANES, LANES)] = acc

    pltpu.sync_copy(out_v, out_hbm.at[pl.ds(wid * bpw, bpw)])


def kernel(user_ids, item_ids, user_hidden_emb, item_hidden_emb):
    b = user_ids.shape[0]
    d = user_hidden_emb.shape[1]
    bpw = b // NW
    n_chunks = bpw // CHUNK

    uid = user_ids.astype(jnp.int32).reshape(NW, n_chunks, CHUNK)
    iid = item_ids.astype(jnp.int32).reshape(NW, n_chunks, CHUNK)

    mesh = plsc.VectorSubcoreMesh(core_axis_name="c", subcore_axis_name="s")
    f = pl.kernel(
        _cf_body,
        out_type=jax.ShapeDtypeStruct((b,), jnp.float32),
        mesh=mesh,
        compiler_params=pltpu.CompilerParams(needs_layout_passes=False),
        scratch_types=[
            pltpu.VMEM((n_chunks, CHUNK), jnp.int32),
            pltpu.VMEM((n_chunks, CHUNK), jnp.int32),
            pltpu.VMEM((CHUNK, d), jnp.float32),
            pltpu.VMEM((CHUNK, d), jnp.float32),
            pltpu.VMEM((bpw,), jnp.float32),
            pltpu.SemaphoreType.DMA,
            pltpu.SemaphoreType.DMA,
        ],
    )
    out = f(uid, iid, user_hidden_emb, item_hidden_emb)
    return out.reshape(-1, 1)


# speedup vs baseline: 2.9500x; 2.9500x over previous
import jax
import jax.numpy as jnp
from jax import lax
from jax.experimental import pallas as pl
from jax.experimental.pallas import tpu as pltpu
from jax.experimental.pallas import tpu_sc as plsc

NC = 2
NS = 16
LANES = 16
NW = NC * NS

CHUNK = 128


def _dot16(ubuf, ibuf, row, d):
    acc = None
    for c in range(d // LANES):
        u = ubuf[row, pl.ds(c * LANES, LANES)]
        v = ibuf[row, pl.ds(c * LANES, LANES)]
        t = u * v
        acc = t if acc is None else acc + t
    return plsc.cumsum(acc)


def _cf_body(uid_hbm, iid_hbm, utab_hbm, itab_hbm, out_hbm,
             uidx_v, iidx_v, ubuf0, ibuf0, ubuf1, ibuf1, out_v,
             sem_u0, sem_i0, sem_u1, sem_i1):
    d = utab_hbm.shape[1]
    n_chunks = uidx_v.shape[0]
    bpw = n_chunks * CHUNK
    wid = lax.axis_index("s") * NC + lax.axis_index("c")

    pltpu.sync_copy(uid_hbm.at[wid], uidx_v)
    pltpu.sync_copy(iid_hbm.at[wid], iidx_v)

    bufs = [(ubuf0, ibuf0, sem_u0, sem_i0), (ubuf1, ibuf1, sem_u1, sem_i1)]

    def start(j):
        ub, ib, su, si = bufs[j % 2]
        cu = pltpu.async_copy(utab_hbm.at[uidx_v.at[j]], ub, su)
        ci = pltpu.async_copy(itab_hbm.at[iidx_v.at[j]], ib, si)
        return cu, ci

    lane = lax.iota(jnp.int32, LANES)
    inflight = {0: start(0)}

    for j in range(n_chunks):
        if j + 1 < n_chunks:
            inflight[j + 1] = start(j + 1)
        cu, ci = inflight.pop(j)
        cu.wait()
        ci.wait()
        ub, ib, _, _ = bufs[j % 2]

        @plsc.parallel_loop(0, CHUNK, unroll=4)
        def _row(r, ub=ub, ib=ib, j=j):
            scan = _dot16(ub, ib, r, d)
            idx = jnp.full((LANES,), j * CHUNK, jnp.int32) + r
            plsc.store_scatter(out_v, [idx], scan, mask=lane == LANES - 1)

    pltpu.sync_copy(out_v, out_hbm.at[pl.ds(wid * bpw, bpw)])


def kernel(user_ids, item_ids, user_hidden_emb, item_hidden_emb):
    b = user_ids.shape[0]
    d = user_hidden_emb.shape[1]
    bpw = b // NW
    n_chunks = bpw // CHUNK

    uid = user_ids.astype(jnp.int32).reshape(NW, n_chunks, CHUNK)
    iid = item_ids.astype(jnp.int32).reshape(NW, n_chunks, CHUNK)

    mesh = plsc.VectorSubcoreMesh(core_axis_name="c", subcore_axis_name="s")
    f = pl.kernel(
        _cf_body,
        out_type=jax.ShapeDtypeStruct((b,), jnp.float32),
        mesh=mesh,
        compiler_params=pltpu.CompilerParams(needs_layout_passes=False),
        scratch_types=[
            pltpu.VMEM((n_chunks, CHUNK), jnp.int32),
            pltpu.VMEM((n_chunks, CHUNK), jnp.int32),
            pltpu.VMEM((CHUNK, d), jnp.float32),
            pltpu.VMEM((CHUNK, d), jnp.float32),
            pltpu.VMEM((CHUNK, d), jnp.float32),
            pltpu.VMEM((CHUNK, d), jnp.float32),
            pltpu.VMEM((bpw,), jnp.float32),
            pltpu.SemaphoreType.DMA,
            pltpu.SemaphoreType.DMA,
            pltpu.SemaphoreType.DMA,
            pltpu.SemaphoreType.DMA,
        ],
    )
    out = f(uid, iid, user_hidden_emb, item_hidden_emb)
    return out.reshape(-1, 1)


# speedup vs baseline: 2.9806x; 1.0104x over previous
import jax
import jax.numpy as jnp
from jax import lax
from jax.experimental import pallas as pl
from jax.experimental.pallas import tpu as pltpu
from jax.experimental.pallas import tpu_sc as plsc

NC = 2
NS = 16
LANES = 16
NW = NC * NS

CHUNK = 128
NBUF = 3


def _dot16(ubuf, ibuf, row, d):
    acc = None
    for c in range(d // LANES):
        u = ubuf[row, pl.ds(c * LANES, LANES)]
        v = ibuf[row, pl.ds(c * LANES, LANES)]
        t = u * v
        acc = t if acc is None else acc + t
    return plsc.cumsum(acc)


def _cf_body(uid_hbm, iid_hbm, utab_hbm, itab_hbm, out_hbm,
             uidx_v, iidx_v, ub0, ib0, ub1, ib1, ub2, ib2, out_v,
             su0, si0, su1, si1, su2, si2):
    ubufs = [ub0, ub1, ub2]
    ibufs = [ib0, ib1, ib2]
    sems_u = [su0, su1, su2]
    sems_i = [si0, si1, si2]
    d = utab_hbm.shape[1]
    bpw = uidx_v.shape[0]
    n_chunks = bpw // CHUNK
    wid = lax.axis_index("s") * NC + lax.axis_index("c")
    base = wid * bpw

    pltpu.sync_copy(uid_hbm.at[pl.ds(base, bpw)], uidx_v)
    pltpu.sync_copy(iid_hbm.at[pl.ds(base, bpw)], iidx_v)

    lane = lax.iota(jnp.int32, LANES)
    zero16 = jnp.zeros((LANES,), jnp.int32)
    out_mask = lane == LANES - 1

    def start(j):
        s = j % NBUF
        cu = pltpu.async_copy(
            utab_hbm.at[uidx_v.at[pl.ds(j * CHUNK, CHUNK)]], ubufs[s],
            sems_u[s])
        ci = pltpu.async_copy(
            itab_hbm.at[iidx_v.at[pl.ds(j * CHUNK, CHUNK)]], ibufs[s],
            sems_i[s])
        return cu, ci

    inflight = {}
    for j in range(min(NBUF, n_chunks)):
        inflight[j] = start(j)

    for j in range(n_chunks):
        cu, ci = inflight.pop(j)
        cu.wait()
        ci.wait()
        s = j % NBUF

        @plsc.parallel_loop(0, CHUNK, unroll=4)
        def _row(r, s=s, j=j):
            scan = _dot16(ubufs[s], ibufs[s], r, d)
            idx = jnp.full((LANES,), j * CHUNK, jnp.int32) + r
            plsc.store_scatter(out_v, [idx], scan, mask=out_mask)

        if j + NBUF < n_chunks:
            inflight[j + NBUF] = start(j + NBUF)

    pltpu.sync_copy(out_v, out_hbm.at[pl.ds(base, bpw)])


def kernel(user_ids, item_ids, user_hidden_emb, item_hidden_emb):
    b = user_ids.shape[0]
    d = user_hidden_emb.shape[1]
    bpw = b // NW

    mesh = plsc.VectorSubcoreMesh(core_axis_name="c", subcore_axis_name="s")
    f = pl.kernel(
        _cf_body,
        out_type=jax.ShapeDtypeStruct((b,), jnp.float32),
        mesh=mesh,
        compiler_params=pltpu.CompilerParams(needs_layout_passes=False),
        scratch_types=[
            pltpu.VMEM((bpw,), jnp.int32),
            pltpu.VMEM((bpw,), jnp.int32),
            pltpu.VMEM((CHUNK, d), jnp.float32),
            pltpu.VMEM((CHUNK, d), jnp.float32),
            pltpu.VMEM((CHUNK, d), jnp.float32),
            pltpu.VMEM((CHUNK, d), jnp.float32),
            pltpu.VMEM((CHUNK, d), jnp.float32),
            pltpu.VMEM((CHUNK, d), jnp.float32),
            pltpu.VMEM((bpw,), jnp.float32),
            pltpu.SemaphoreType.DMA,
            pltpu.SemaphoreType.DMA,
            pltpu.SemaphoreType.DMA,
            pltpu.SemaphoreType.DMA,
            pltpu.SemaphoreType.DMA,
            pltpu.SemaphoreType.DMA,
        ],
    )
    out = f(user_ids.astype(jnp.int32), item_ids.astype(jnp.int32),
            user_hidden_emb, item_hidden_emb)
    return out.reshape(-1, 1)


# speedup vs baseline: 3.0778x; 1.0326x over previous
import jax
import jax.numpy as jnp
from jax import lax
from jax.experimental import pallas as pl
from jax.experimental.pallas import tpu as pltpu
from jax.experimental.pallas import tpu_sc as plsc

NC = 2
NS = 16
LANES = 16
NW = NC * NS

CHUNK = 128
NBUF = 3


def _dot16(ubuf, ibuf, row, d):
    acc = None
    for c in range(d // LANES):
        u = ubuf[row, pl.ds(c * LANES, LANES)]
        v = ibuf[row, pl.ds(c * LANES, LANES)]
        t = u * v
        acc = t if acc is None else acc + t
    return plsc.cumsum(acc)


def _cf_body(uid_hbm, iid_hbm, utab_hbm, itab_hbm, out_hbm,
             uidx_v, iidx_v, ub0, ib0, ub1, ib1, ub2, ib2, out_v,
             su0, si0, su1, si1, su2, si2):
    ubufs = [ub0, ub1, ub2]
    ibufs = [ib0, ib1, ib2]
    sems_u = [su0, su1, su2]
    sems_i = [si0, si1, si2]
    d = utab_hbm.shape[1]
    bpw = uidx_v.shape[0]
    n_chunks = bpw // CHUNK
    wid = lax.axis_index("s") * NC + lax.axis_index("c")
    base = wid * bpw

    pltpu.sync_copy(uid_hbm.at[pl.ds(base, bpw)], uidx_v)
    pltpu.sync_copy(iid_hbm.at[pl.ds(base, bpw)], iidx_v)

    lane = lax.iota(jnp.int32, LANES)
    zero16 = jnp.zeros((LANES,), jnp.int32)
    out_mask = lane == LANES - 1

    def start(j):
        s = j % NBUF
        cu = pltpu.async_copy(
            utab_hbm.at[uidx_v.at[pl.ds(j * CHUNK, CHUNK)]], ubufs[s],
            sems_u[s])
        ci = pltpu.async_copy(
            itab_hbm.at[iidx_v.at[pl.ds(j * CHUNK, CHUNK)]], ibufs[s],
            sems_i[s])
        return cu, ci

    inflight = {}
    for j in range(min(NBUF, n_chunks)):
        inflight[j] = start(j)

    for j in range(n_chunks):
        cu, ci = inflight.pop(j)
        cu.wait()
        ci.wait()
        s = j % NBUF

        @plsc.parallel_loop(0, CHUNK, unroll=2)
        def _row(r, s=s, j=j):
            scan = _dot16(ubufs[s], ibufs[s], r, d)
            idx = jnp.full((LANES,), j * CHUNK, jnp.int32) + r
            plsc.store_scatter(out_v, [idx], scan, mask=out_mask)

        if j + NBUF < n_chunks:
            inflight[j + NBUF] = start(j + NBUF)

    pltpu.sync_copy(out_v, out_hbm.at[pl.ds(base, bpw)])


def kernel(user_ids, item_ids, user_hidden_emb, item_hidden_emb):
    b = user_ids.shape[0]
    d = user_hidden_emb.shape[1]
    bpw = b // NW

    mesh = plsc.VectorSubcoreMesh(core_axis_name="c", subcore_axis_name="s")
    f = pl.kernel(
        _cf_body,
        out_type=jax.ShapeDtypeStruct((b,), jnp.float32),
        mesh=mesh,
        compiler_params=pltpu.CompilerParams(needs_layout_passes=False),
        scratch_types=[
            pltpu.VMEM((bpw,), jnp.int32),
            pltpu.VMEM((bpw,), jnp.int32),
            pltpu.VMEM((CHUNK, d), jnp.float32),
            pltpu.VMEM((CHUNK, d), jnp.float32),
            pltpu.VMEM((CHUNK, d), jnp.float32),
            pltpu.VMEM((CHUNK, d), jnp.float32),
            pltpu.VMEM((CHUNK, d), jnp.float32),
            pltpu.VMEM((CHUNK, d), jnp.float32),
            pltpu.VMEM((bpw,), jnp.float32),
            pltpu.SemaphoreType.DMA,
            pltpu.SemaphoreType.DMA,
            pltpu.SemaphoreType.DMA,
            pltpu.SemaphoreType.DMA,
            pltpu.SemaphoreType.DMA,
            pltpu.SemaphoreType.DMA,
        ],
    )
    out = f(user_ids.astype(jnp.int32), item_ids.astype(jnp.int32),
            user_hidden_emb, item_hidden_emb)
    return out.reshape(-1, 1)
